# Initial kernel scaffold; baseline (speedup 1.0000x reference)
import jax
import jax.numpy as jnp
from jax.experimental import pallas as pl


def kernel(x_Persona, x_Ubicacion, edge_index_visits, edge_index_rev, W1v_l, b1v, W1v_r, W1r_l, b1r, W1r_r, W2v_l, b2v, W2v_r, W2r_l, b2r, W2r_r):
    raise NotImplementedError("write your pallas kernel here")


# speedup vs baseline: 11.6139x; 11.6139x over previous
import functools

import jax
import jax.numpy as jnp
from jax import lax
from jax.experimental import pallas as pl
from jax.experimental.pallas import tpu as pltpu
from jax.experimental.pallas import tpu_sc as plsc

N = 10000
D = 128
H = 16
E = 320000
NC = 2
NS = 16
NW = NC * NS
CHUNK = 125
EPT = E // NW
NCH = EPT // CHUNK
RPT = N // NS
RCH = RPT // CHUNK

_F32 = jnp.float32


def _make_seg(with_counts):
  n_out = 4 if with_counts else 2
  outs = [jax.ShapeDtypeStruct((NC, N, H), _F32)] * n_out
  scratch = [
      pltpu.VMEM((NCH, CHUNK), jnp.int32),
      pltpu.VMEM((NCH, CHUNK), jnp.int32),
      pltpu.VMEM((CHUNK, H), _F32),
      pltpu.VMEM((CHUNK, H), _F32),
      pltpu.VMEM_SHARED((N, H), _F32),
      pltpu.VMEM_SHARED((N, H), _F32),
      pltpu.SemaphoreType.DMA,
      pltpu.SemaphoreType.DMA,
  ]
  if with_counts:
    scratch += [
        pltpu.VMEM((CHUNK, H), _F32),
        pltpu.VMEM_SHARED((N, H), _F32),
        pltpu.VMEM_SHARED((N, H), _F32),
    ]
  mesh = plsc.VectorSubcoreMesh(core_axis_name="c", subcore_axis_name="s")

  @functools.partial(
      pl.kernel, mesh=mesh, out_type=outs, scratch_types=scratch,
      compiler_params=pltpu.CompilerParams(use_tc_tiling_on_sc=False))
  def seg(*refs):
    if with_counts:
      (yv, yr, svi, dvi, sri, dri, out_sv, out_sr, out_cv, out_cr,
       sidx, didx, rowsA, rowsB, acc_v, acc_r, semA, semB,
       ones_r, accc_v, accc_r) = refs
    else:
      (yv, yr, svi, dvi, sri, dri, out_sv, out_sr,
       sidx, didx, rowsA, rowsB, acc_v, acc_r, semA, semB) = refs
      accc_v = accc_r = ones_r = None

    c = lax.axis_index("c")
    s = lax.axis_index("s")
    wid = c * NS + s
    row0 = s * RPT

    zrow = jnp.zeros((H,), _F32)
    for i in range(CHUNK):
      rowsA[i] = zrow
    if with_counts:
      orow = jnp.ones((H,), _F32)
      for i in range(CHUNK):
        ones_r[i] = orow
    accs = [acc_v, acc_r] + ([accc_v, accc_r] if with_counts else [])
    for acc in accs:
      for k in range(RCH):
        pltpu.sync_copy(rowsA, acc.at[pl.ds(row0 + k * CHUNK, CHUNK)])
    plsc.subcore_barrier()

    ebase = wid * NCH
    work = [(svi, dvi, yv, acc_v, accc_v), (sri, dri, yr, acc_r, accc_r)]
    for (si_h, di_h, y_h, acc, accc) in work:
      pltpu.sync_copy(si_h.at[pl.ds(ebase, NCH)], sidx)
      pltpu.sync_copy(di_h.at[pl.ds(ebase, NCH)], didx)
      pltpu.async_copy(y_h.at[sidx.at[0]], rowsA, semA)

      def pair(i, _, y_h=y_h, acc=acc, accc=accc):
        j0 = 2 * i
        j1 = j0 + 1
        pltpu.make_async_copy(y_h.at[sidx.at[j0]], rowsA, semA).wait()
        pltpu.async_copy(y_h.at[sidx.at[j1]], rowsB, semB)
        pltpu.sync_copy(rowsA, acc.at[didx.at[j0]], add=True)
        if with_counts:
          pltpu.sync_copy(ones_r, accc.at[didx.at[j0]], add=True)
        pltpu.make_async_copy(y_h.at[sidx.at[j1]], rowsB, semB).wait()

        @pl.when(j0 + 2 < NCH)
        def _fire():
          pltpu.async_copy(y_h.at[sidx.at[j0 + 2]], rowsA, semA)

        pltpu.sync_copy(rowsB, acc.at[didx.at[j1]], add=True)
        if with_counts:
          pltpu.sync_copy(ones_r, accc.at[didx.at[j1]], add=True)
        return 0

      lax.fori_loop(0, NCH // 2, pair, 0)

    plsc.subcore_barrier()
    pairs = [(acc_v, out_sv), (acc_r, out_sr)]
    if with_counts:
      pairs += [(accc_v, out_cv), (accc_r, out_cr)]
    for acc, out in pairs:
      for k in range(RCH):
        r = row0 + k * CHUNK
        pltpu.sync_copy(acc.at[pl.ds(r, CHUNK)], rowsA)
        pltpu.sync_copy(rowsA, out.at[c, pl.ds(r, CHUNK)])

  return seg


_seg_counts = _make_seg(True)
_seg_plain = _make_seg(False)


_R = 2000


def _proj1_body(xp, xu, wvl, wrr, wrl, wvr, yv, rr, yr, rv):
  p = xp[...]
  u = xu[...]
  yv[...] = jnp.dot(p, wvl[...], preferred_element_type=_F32)
  rr[...] = jnp.dot(p, wrr[...], preferred_element_type=_F32)
  yr[...] = jnp.dot(u, wrl[...], preferred_element_type=_F32)
  rv[...] = jnp.dot(u, wvr[...], preferred_element_type=_F32)


def _proj1(xp, xu, wvl, wrr, wrl, wvr):
  bx = pl.BlockSpec((_R, D), lambda i: (i, 0))
  bw = pl.BlockSpec((D, H), lambda i: (0, 0))
  bo = pl.BlockSpec((_R, H), lambda i: (i, 0))
  return pl.pallas_call(
      _proj1_body,
      grid=(N // _R,),
      in_specs=[bx, bx, bw, bw, bw, bw],
      out_specs=[bo] * 4,
      out_shape=[jax.ShapeDtypeStruct((N, H), _F32)] * 4,
  )(xp, xu, wvl, wrr, wrl, wvr)


def _mid_body(sv, cv, rv, sr, cr, rr, b1v, b1r, wvl, wvr, wrl, wrr,
              yv2, yr2, rv2, rr2, rcpv, rcpr):
  cva = cv[...]
  cra = cr[...]
  sva = sv[...]
  sra = sr[...]
  rcv = 1.0 / jnp.maximum(cva[0] + cva[1], 1.0)
  rcr = 1.0 / jnp.maximum(cra[0] + cra[1], 1.0)
  u1 = jnp.maximum((sva[0] + sva[1]) * rcv + b1v[...] + rv[...], 0.0)
  p1 = jnp.maximum((sra[0] + sra[1]) * rcr + b1r[...] + rr[...], 0.0)
  yv2[...] = jnp.dot(p1, wvl[...], preferred_element_type=_F32)
  yr2[...] = jnp.dot(u1, wrl[...], preferred_element_type=_F32)
  rv2[...] = jnp.dot(u1, wvr[...], preferred_element_type=_F32)
  rr2[...] = jnp.dot(p1, wrr[...], preferred_element_type=_F32)
  rcpv[...] = rcv
  rcpr[...] = rcr


def _mid(sv, cv, rv, sr, cr, rr, b1v, b1r, wvl, wvr, wrl, wrr):
  bp = pl.BlockSpec((NC, _R, H), lambda i: (0, i, 0))
  bn = pl.BlockSpec((_R, H), lambda i: (i, 0))
  bb = pl.BlockSpec((1, H), lambda i: (0, 0))
  bw = pl.BlockSpec((H, H), lambda i: (0, 0))
  return pl.pallas_call(
      _mid_body,
      grid=(N // _R,),
      in_specs=[bp, bp, bn, bp, bp, bn, bb, bb, bw, bw, bw, bw],
      out_specs=[bn] * 6,
      out_shape=[jax.ShapeDtypeStruct((N, H), _F32)] * 6,
  )(sv, cv, rv, sr, cr, rr, b1v, b1r, wvl, wvr, wrl, wrr)


def _fin_body(sv2, rcpv, rv2, b2v, sr2, rcpr, rr2, b2r, p2, u2):
  sva = sv2[...]
  sra = sr2[...]
  u2[...] = jnp.maximum(
      (sva[0] + sva[1]) * rcpv[...] + b2v[...] + rv2[...], 0.0)
  p2[...] = jnp.maximum(
      (sra[0] + sra[1]) * rcpr[...] + b2r[...] + rr2[...], 0.0)


def _fin(sv2, rcpv, rv2, b2v, sr2, rcpr, rr2, b2r):
  bp = pl.BlockSpec((NC, _R, H), lambda i: (0, i, 0))
  bn = pl.BlockSpec((_R, H), lambda i: (i, 0))
  bb = pl.BlockSpec((1, H), lambda i: (0, 0))
  return pl.pallas_call(
      _fin_body,
      grid=(N // _R,),
      in_specs=[bp, bn, bn, bb, bp, bn, bn, bb],
      out_specs=[bn, bn],
      out_shape=[jax.ShapeDtypeStruct((N, H), _F32)] * 2,
  )(sv2, rcpv, rv2, b2v, sr2, rcpr, rr2, b2r)


def kernel(x_Persona, x_Ubicacion, edge_index_visits, edge_index_rev,
           W1v_l, b1v, W1v_r, W1r_l, b1r, W1r_r,
           W2v_l, b2v, W2v_r, W2r_l, b2r, W2r_r):
  svi = edge_index_visits[0].reshape(E // CHUNK, CHUNK)
  dvi = edge_index_visits[1].reshape(E // CHUNK, CHUNK)
  sri = edge_index_rev[0].reshape(E // CHUNK, CHUNK)
  dri = edge_index_rev[1].reshape(E // CHUNK, CHUNK)

  yv, rr1, yr, rv1 = _proj1(x_Persona, x_Ubicacion, W1v_l, W1r_r, W1r_l,
                            W1v_r)
  sv, sr, cv, cr = _seg_counts(yv, yr, svi, dvi, sri, dri)
  yv2, yr2, rv2, rr2, rcpv, rcpr = _mid(
      sv, cv, rv1, sr, cr, rr1, b1v.reshape(1, H), b1r.reshape(1, H),
      W2v_l, W2v_r, W2r_l, W2r_r)
  sv2, sr2 = _seg_plain(yv2, yr2, svi, dvi, sri, dri)
  p2, u2 = _fin(sv2, rcpv, rv2, b2v.reshape(1, H), sr2, rcpr, rr2,
                b2r.reshape(1, H))
  return (p2, u2)


# speedup vs baseline: 17.0997x; 1.4723x over previous
import functools

import jax
import jax.numpy as jnp
from jax import lax
from jax.experimental import pallas as pl
from jax.experimental.pallas import tpu as pltpu
from jax.experimental.pallas import tpu_sc as plsc

N = 10000
D = 128
H = 16
E = 320000
NC = 2
NS = 16
NW = NC * NS
CHUNK = 125
EPT = E // NW
NCH = EPT // CHUNK
RPT = N // NS
RCH = RPT // CHUNK

_F32 = jnp.float32


def _make_seg(with_counts):
  n_out = 4 if with_counts else 2
  outs = [jax.ShapeDtypeStruct((NC, N, H), _F32)] * n_out
  NB = 8
  scratch = [
      pltpu.VMEM((NCH, CHUNK), jnp.int32),
      pltpu.VMEM((NCH, CHUNK), jnp.int32),
      pltpu.VMEM_SHARED((N, H), _F32),
      pltpu.VMEM_SHARED((N, H), _F32),
  ]
  scratch += [pltpu.VMEM((CHUNK, H), _F32)] * NB
  scratch += [pltpu.SemaphoreType.DMA] * NB
  scratch += [pltpu.SemaphoreType.DMA] * NB
  if with_counts:
    scratch += [
        pltpu.VMEM((CHUNK, H), _F32),
        pltpu.VMEM_SHARED((N, H), _F32),
        pltpu.VMEM_SHARED((N, H), _F32),
        pltpu.SemaphoreType.DMA,
    ] + [pltpu.SemaphoreType.DMA] * (NB - 1)
  mesh = plsc.VectorSubcoreMesh(core_axis_name="c", subcore_axis_name="s")

  @functools.partial(
      pl.kernel, mesh=mesh, out_type=outs, scratch_types=scratch,
      compiler_params=pltpu.CompilerParams(use_tc_tiling_on_sc=False))
  def seg(*refs):
    if with_counts:
      (yv, yr, svi, dvi, sri, dri, out_sv, out_sr, out_cv, out_cr,
       sidx, didx, acc_v, acc_r, *rest) = refs
      bufs = rest[:NB]
      gsem = rest[NB:2 * NB]
      ssem = rest[2 * NB:3 * NB]
      ones_r = rest[3 * NB]
      accc_v = rest[3 * NB + 1]
      accc_r = rest[3 * NB + 2]
      csem = rest[3 * NB + 3:3 * NB + 3 + NB]
    else:
      (yv, yr, svi, dvi, sri, dri, out_sv, out_sr,
       sidx, didx, acc_v, acc_r, *rest) = refs
      bufs = rest[:NB]
      gsem = rest[NB:2 * NB]
      ssem = rest[2 * NB:3 * NB]
      accc_v = accc_r = ones_r = csem = None

    c = lax.axis_index("c")
    s = lax.axis_index("s")
    wid = c * NS + s
    row0 = s * RPT

    zrow = jnp.zeros((H,), _F32)
    for i in range(CHUNK):
      bufs[0][i] = zrow
    if with_counts:
      orow = jnp.ones((H,), _F32)
      for i in range(CHUNK):
        ones_r[i] = orow
    accs = [acc_v, acc_r] + ([accc_v, accc_r] if with_counts else [])
    for acc in accs:
      for k in range(RCH):
        pltpu.sync_copy(bufs[0], acc.at[pl.ds(row0 + k * CHUNK, CHUNK)])
    plsc.subcore_barrier()

    ebase = wid * NCH
    work = [(svi, dvi, yv, acc_v, accc_v), (sri, dri, yr, acc_r, accc_r)]
    for (si_h, di_h, y_h, acc, accc) in work:
      pltpu.sync_copy(si_h.at[pl.ds(ebase, NCH)], sidx)
      pltpu.sync_copy(di_h.at[pl.ds(ebase, NCH)], didx)

      if with_counts:
        def cnt_step(j, _, accc=accc):
          pltpu.sync_copy(ones_r, accc.at[didx.at[j]], add=True)
          return 0

        lax.fori_loop(0, NCH, cnt_step, 0)

      for k in range(NB // 2):
        pltpu.async_copy(y_h.at[sidx.at[k]], bufs[k], gsem[k])

      def step(i, _, y_h=y_h, acc=acc):
        for k in range(NB):
          j = NB * i + k
          pltpu.make_async_copy(y_h.at[sidx.at[j]], bufs[k], gsem[k]).wait()
          pltpu.async_copy(bufs[k], acc.at[didx.at[j]], ssem[k], add=True)
          b2 = (k + NB // 2) % NB

          @pl.when(j + NB // 2 < NCH)
          def _next():
            @pl.when(j >= NB // 2)
            def _drain():
              pltpu.make_async_copy(
                  bufs[b2], acc.at[didx.at[j - NB // 2]], ssem[b2]).wait()

            pltpu.async_copy(
                y_h.at[sidx.at[j + NB // 2]], bufs[b2], gsem[b2])

        return 0

      lax.fori_loop(0, NCH // NB, step, 0)

      for k in range(NB):
        j = NCH - NB + k
        pltpu.make_async_copy(
            bufs[k], acc.at[didx.at[j]], ssem[k]).wait()

    plsc.subcore_barrier()
    pairs = [(acc_v, out_sv), (acc_r, out_sr)]
    if with_counts:
      pairs += [(accc_v, out_cv), (accc_r, out_cr)]
    for acc, out in pairs:
      for k in range(RCH):
        r = row0 + k * CHUNK
        pltpu.sync_copy(acc.at[pl.ds(r, CHUNK)], bufs[k % NB])
        pltpu.sync_copy(bufs[k % NB], out.at[c, pl.ds(r, CHUNK)])

  return seg


_seg_counts = _make_seg(True)
_seg_plain = _make_seg(False)


_R = 2000


def _proj1_body(xp, xu, wvl, wrr, wrl, wvr, yv, rr, yr, rv):
  p = xp[...]
  u = xu[...]
  yv[...] = jnp.dot(p, wvl[...], preferred_element_type=_F32)
  rr[...] = jnp.dot(p, wrr[...], preferred_element_type=_F32)
  yr[...] = jnp.dot(u, wrl[...], preferred_element_type=_F32)
  rv[...] = jnp.dot(u, wvr[...], preferred_element_type=_F32)


def _proj1(xp, xu, wvl, wrr, wrl, wvr):
  bx = pl.BlockSpec((_R, D), lambda i: (i, 0))
  bw = pl.BlockSpec((D, H), lambda i: (0, 0))
  bo = pl.BlockSpec((_R, H), lambda i: (i, 0))
  return pl.pallas_call(
      _proj1_body,
      grid=(N // _R,),
      in_specs=[bx, bx, bw, bw, bw, bw],
      out_specs=[bo] * 4,
      out_shape=[jax.ShapeDtypeStruct((N, H), _F32)] * 4,
  )(xp, xu, wvl, wrr, wrl, wvr)


def _mid_body(sv, cv, rv, sr, cr, rr, b1v, b1r, wvl, wvr, wrl, wrr,
              yv2, yr2, rv2, rr2, rcpv, rcpr):
  cva = cv[...]
  cra = cr[...]
  sva = sv[...]
  sra = sr[...]
  rcv = 1.0 / jnp.maximum(cva[0] + cva[1], 1.0)
  rcr = 1.0 / jnp.maximum(cra[0] + cra[1], 1.0)
  u1 = jnp.maximum((sva[0] + sva[1]) * rcv + b1v[...] + rv[...], 0.0)
  p1 = jnp.maximum((sra[0] + sra[1]) * rcr + b1r[...] + rr[...], 0.0)
  yv2[...] = jnp.dot(p1, wvl[...], preferred_element_type=_F32)
  yr2[...] = jnp.dot(u1, wrl[...], preferred_element_type=_F32)
  rv2[...] = jnp.dot(u1, wvr[...], preferred_element_type=_F32)
  rr2[...] = jnp.dot(p1, wrr[...], preferred_element_type=_F32)
  rcpv[...] = rcv
  rcpr[...] = rcr


def _mid(sv, cv, rv, sr, cr, rr, b1v, b1r, wvl, wvr, wrl, wrr):
  bp = pl.BlockSpec((NC, _R, H), lambda i: (0, i, 0))
  bn = pl.BlockSpec((_R, H), lambda i: (i, 0))
  bb = pl.BlockSpec((1, H), lambda i: (0, 0))
  bw = pl.BlockSpec((H, H), lambda i: (0, 0))
  return pl.pallas_call(
      _mid_body,
      grid=(N // _R,),
      in_specs=[bp, bp, bn, bp, bp, bn, bb, bb, bw, bw, bw, bw],
      out_specs=[bn] * 6,
      out_shape=[jax.ShapeDtypeStruct((N, H), _F32)] * 6,
  )(sv, cv, rv, sr, cr, rr, b1v, b1r, wvl, wvr, wrl, wrr)


def _fin_body(sv2, rcpv, rv2, b2v, sr2, rcpr, rr2, b2r, p2, u2):
  sva = sv2[...]
  sra = sr2[...]
  u2[...] = jnp.maximum(
      (sva[0] + sva[1]) * rcpv[...] + b2v[...] + rv2[...], 0.0)
  p2[...] = jnp.maximum(
      (sra[0] + sra[1]) * rcpr[...] + b2r[...] + rr2[...], 0.0)


def _fin(sv2, rcpv, rv2, b2v, sr2, rcpr, rr2, b2r):
  bp = pl.BlockSpec((NC, _R, H), lambda i: (0, i, 0))
  bn = pl.BlockSpec((_R, H), lambda i: (i, 0))
  bb = pl.BlockSpec((1, H), lambda i: (0, 0))
  return pl.pallas_call(
      _fin_body,
      grid=(N // _R,),
      in_specs=[bp, bn, bn, bb, bp, bn, bn, bb],
      out_specs=[bn, bn],
      out_shape=[jax.ShapeDtypeStruct((N, H), _F32)] * 2,
  )(sv2, rcpv, rv2, b2v, sr2, rcpr, rr2, b2r)


def kernel(x_Persona, x_Ubicacion, edge_index_visits, edge_index_rev,
           W1v_l, b1v, W1v_r, W1r_l, b1r, W1r_r,
           W2v_l, b2v, W2v_r, W2r_l, b2r, W2r_r):
  svi = edge_index_visits[0].reshape(E // CHUNK, CHUNK)
  dvi = edge_index_visits[1].reshape(E // CHUNK, CHUNK)
  sri = edge_index_rev[0].reshape(E // CHUNK, CHUNK)
  dri = edge_index_rev[1].reshape(E // CHUNK, CHUNK)

  yv, rr1, yr, rv1 = _proj1(x_Persona, x_Ubicacion, W1v_l, W1r_r, W1r_l,
                            W1v_r)
  sv, sr, cv, cr = _seg_counts(yv, yr, svi, dvi, sri, dri)
  yv2, yr2, rv2, rr2, rcpv, rcpr = _mid(
      sv, cv, rv1, sr, cr, rr1, b1v.reshape(1, H), b1r.reshape(1, H),
      W2v_l, W2v_r, W2r_l, W2r_r)
  sv2, sr2 = _seg_plain(yv2, yr2, svi, dvi, sri, dri)
  p2, u2 = _fin(sv2, rcpv, rv2, b2v.reshape(1, H), sr2, rcpr, rr2,
                b2r.reshape(1, H))
  return (p2, u2)


# speedup vs baseline: 18.4533x; 1.0792x over previous
import functools

import jax
import jax.numpy as jnp
from jax import lax
from jax.experimental import pallas as pl
from jax.experimental.pallas import tpu as pltpu
from jax.experimental.pallas import tpu_sc as plsc

N = 10000
D = 128
H = 16
E = 320000
NC = 2
NS = 16
CHUNK = 125
NCHT = E // (NS * CHUNK)
RPT = N // NS
RCH = RPT // CHUNK
NB = 8

_F32 = jnp.float32


def _sc_scratch():
  s = [
      pltpu.VMEM((NCHT, CHUNK), jnp.int32),
      pltpu.VMEM((NCHT, CHUNK), jnp.int32),
      pltpu.VMEM_SHARED((N, H), _F32),
  ]
  s += [pltpu.VMEM((CHUNK, H), _F32)] * NB
  s += [pltpu.SemaphoreType.DMA] * NB
  s += [pltpu.SemaphoreType.DMA] * NB
  return s


def _zero_stripe(buf, accs, row0):
  zrow = jnp.zeros((H,), _F32)
  for i in range(CHUNK):
    buf[i] = zrow
  for acc in accs:
    for k in range(RCH):
      pltpu.sync_copy(buf, acc.at[pl.ds(row0 + k * CHUNK, CHUNK)])


def _scatter_loop(y_flat, sidx, didx, acc, bufs, gsem, ssem):
  for k in range(NB // 2):
    pltpu.async_copy(y_flat.at[sidx.at[k]], bufs[k], gsem[k])

  def step(i, _):
    for k in range(NB):
      j = NB * i + k
      pltpu.make_async_copy(y_flat.at[sidx.at[j]], bufs[k], gsem[k]).wait()
      pltpu.async_copy(bufs[k], acc.at[didx.at[j]], ssem[k], add=True)
      b2 = (k + NB // 2) % NB

      @pl.when(j + NB // 2 < NCHT)
      def _next():
        @pl.when(j >= NB // 2)
        def _drain():
          pltpu.make_async_copy(
              bufs[b2], acc.at[didx.at[j - NB // 2]], ssem[b2]).wait()

        pltpu.async_copy(
            y_flat.at[sidx.at[j + NB // 2]], bufs[b2], gsem[b2])

    return 0

  lax.fori_loop(0, NCHT // NB, step, 0)
  for k in range(NB):
    j = NCHT - NB + k
    pltpu.make_async_copy(bufs[k], acc.at[didx.at[j]], ssem[k]).wait()


_MESH = plsc.VectorSubcoreMesh(core_axis_name="c", subcore_axis_name="s")
_SC_PARAMS = pltpu.CompilerParams(use_tc_tiling_on_sc=False)


@functools.partial(
    pl.kernel, mesh=_MESH,
    out_type=[jax.ShapeDtypeStruct((NC, N, H), _F32),
              jax.ShapeDtypeStruct((NC, N, H), _F32)],
    scratch_types=_sc_scratch() + [
        pltpu.VMEM((CHUNK, H), _F32),
        pltpu.VMEM_SHARED((N, H), _F32),
    ],
    compiler_params=_SC_PARAMS)
def _sc1(y1, srcs, dsts, out_s, out_rcp, sidx, didx, acc, *rest):
  bufs = rest[:NB]
  gsem = rest[NB:2 * NB]
  ssem = rest[2 * NB:3 * NB]
  ones_r = rest[3 * NB]
  acc_c = rest[3 * NB + 1]

  c = lax.axis_index("c")
  s = lax.axis_index("s")
  row0 = s * RPT

  _zero_stripe(bufs[0], [acc, acc_c], row0)
  orow = jnp.ones((H,), _F32)
  for i in range(CHUNK):
    ones_r[i] = orow
  plsc.subcore_barrier()

  ebase = s * NCHT
  pltpu.sync_copy(srcs.at[c, pl.ds(ebase, NCHT)], sidx)
  pltpu.sync_copy(dsts.at[c, pl.ds(ebase, NCHT)], didx)

  def cnt_step(j, _):
    pltpu.sync_copy(ones_r, acc_c.at[didx.at[j]], add=True)
    return 0

  lax.fori_loop(0, NCHT, cnt_step, 0)
  _scatter_loop(y1, sidx, didx, acc, bufs, gsem, ssem)
  plsc.subcore_barrier()

  for k in range(RCH):
    r = row0 + k * CHUNK
    pltpu.sync_copy(acc.at[pl.ds(r, CHUNK)], bufs[0])
    pltpu.sync_copy(bufs[0], out_s.at[c, pl.ds(r, CHUNK)])
    pltpu.sync_copy(acc_c.at[pl.ds(r, CHUNK)], bufs[1])
    pltpu.sync_copy(bufs[1], out_rcp.at[c, pl.ds(r, CHUNK)])


@functools.partial(
    pl.kernel, mesh=_MESH,
    out_type=[jax.ShapeDtypeStruct((NC, N, H), _F32)],
    scratch_types=_sc_scratch() + [
        pltpu.VMEM((CHUNK, H), _F32),
        pltpu.VMEM((CHUNK, H), _F32),
        pltpu.VMEM((H,), _F32),
    ],
    compiler_params=_SC_PARAMS)
def _sc2(y2, srcs, dsts, rcp, r2, b2, out2, sidx, didx, acc,
         *rest):
  bufs = rest[:NB]
  gsem = rest[NB:2 * NB]
  ssem = rest[2 * NB:3 * NB]
  rbuf = rest[3 * NB]
  tbuf = rest[3 * NB + 1]
  bbuf = rest[3 * NB + 2]

  c = lax.axis_index("c")
  s = lax.axis_index("s")
  row0 = s * RPT

  _zero_stripe(bufs[0], [acc], row0)
  plsc.subcore_barrier()

  ebase = s * NCHT
  pltpu.sync_copy(srcs.at[c, pl.ds(ebase, NCHT)], sidx)
  pltpu.sync_copy(dsts.at[c, pl.ds(ebase, NCHT)], didx)
  pltpu.sync_copy(b2.at[c], bbuf)
  _scatter_loop(y2, sidx, didx, acc, bufs, gsem, ssem)
  plsc.subcore_barrier()

  bias = bbuf[...]
  for k in range(RCH):
    r = row0 + k * CHUNK
    pltpu.sync_copy(acc.at[pl.ds(r, CHUNK)], bufs[0])
    pltpu.sync_copy(rcp.at[c, pl.ds(r, CHUNK)], rbuf)
    pltpu.sync_copy(r2.at[c, pl.ds(r, CHUNK)], tbuf)

    def ep_row(i, _):
      bufs[0][i] = jnp.maximum(
          bufs[0][i] * rbuf[i] + bias + tbuf[i], 0.0)
      return 0

    lax.fori_loop(0, CHUNK, ep_row, 0)

    pltpu.sync_copy(bufs[0], out2.at[c, pl.ds(r, CHUNK)])


_R = 2000


def _tc1_body(xp, xu, wvl, wrl, wvr, wrr, y1, r1):
  p = xp[...]
  u = xu[...]
  y1[0] = jnp.dot(p, wvl[...], preferred_element_type=_F32)
  y1[1] = jnp.dot(u, wrl[...], preferred_element_type=_F32)
  r1[0] = jnp.dot(u, wvr[...], preferred_element_type=_F32)
  r1[1] = jnp.dot(p, wrr[...], preferred_element_type=_F32)


def _tc1(xp, xu, wvl, wrl, wvr, wrr):
  bx = pl.BlockSpec((_R, D), lambda i: (i, 0))
  bw = pl.BlockSpec((D, H), lambda i: (0, 0))
  bo = pl.BlockSpec((NC, _R, H), lambda i: (0, i, 0))
  return pl.pallas_call(
      _tc1_body,
      grid=(N // _R,),
      in_specs=[bx, bx, bw, bw, bw, bw],
      out_specs=[bo, bo],
      out_shape=[jax.ShapeDtypeStruct((NC, N, H), _F32)] * 2,
  )(xp, xu, wvl, wrl, wvr, wrr)


def _tc2_body(s1, cnt, r1, b1, wvl, wrl, wvr, wrr, y2, r2, rcp):
  sa = s1[...]
  ra = 1.0 / jnp.maximum(cnt[...], 1.0)
  oa = r1[...]
  ba = b1[...]
  u1 = jnp.maximum(sa[0] * ra[0] + ba[0] + oa[0], 0.0)
  p1 = jnp.maximum(sa[1] * ra[1] + ba[1] + oa[1], 0.0)
  y2[0] = jnp.dot(p1, wvl[...], preferred_element_type=_F32)
  y2[1] = jnp.dot(u1, wrl[...], preferred_element_type=_F32)
  r2[0] = jnp.dot(u1, wvr[...], preferred_element_type=_F32)
  r2[1] = jnp.dot(p1, wrr[...], preferred_element_type=_F32)
  rcp[...] = ra


def _tc2(s1, rcp, r1, b1, wvl, wrl, wvr, wrr):
  bp = pl.BlockSpec((NC, _R, H), lambda i: (0, i, 0))
  bb = pl.BlockSpec((NC, H), lambda i: (0, 0))
  bw = pl.BlockSpec((H, H), lambda i: (0, 0))
  return pl.pallas_call(
      _tc2_body,
      grid=(N // _R,),
      in_specs=[bp, bp, bp, bb, bw, bw, bw, bw],
      out_specs=[bp, bp, bp],
      out_shape=[jax.ShapeDtypeStruct((NC, N, H), _F32)] * 3,
  )(s1, rcp, r1, b1, wvl, wrl, wvr, wrr)


def kernel(x_Persona, x_Ubicacion, edge_index_visits, edge_index_rev,
           W1v_l, b1v, W1v_r, W1r_l, b1r, W1r_r,
           W2v_l, b2v, W2v_r, W2r_l, b2r, W2r_r):
  nch = E // CHUNK
  srcs = jnp.stack([edge_index_visits[0].reshape(nch, CHUNK),
                    edge_index_rev[0].reshape(nch, CHUNK) + N])
  dsts = jnp.stack([edge_index_visits[1].reshape(nch, CHUNK),
                    edge_index_rev[1].reshape(nch, CHUNK)])
  b1 = jnp.stack([b1v, b1r])
  b2 = jnp.stack([b2v, b2r])

  y1, r1 = _tc1(x_Persona, x_Ubicacion, W1v_l, W1r_l, W1v_r, W1r_r)
  s1, cnt = _sc1(y1.reshape(NC * N, H), srcs, dsts)
  y2, r2, rcp = _tc2(s1, cnt, r1, b1, W2v_l, W2r_l, W2v_r, W2r_r)
  out2 = _sc2(y2.reshape(NC * N, H), srcs, dsts, rcp, r2, b2)[0]
  return (out2[1], out2[0])


# speedup vs baseline: 18.9012x; 1.0243x over previous
import functools

import jax
import jax.numpy as jnp
from jax import lax
from jax.experimental import pallas as pl
from jax.experimental.pallas import tpu as pltpu
from jax.experimental.pallas import tpu_sc as plsc

N = 10000
D = 128
H = 16
E = 320000
NC = 2
NS = 16
CHUNK = 125
NCHT = E // (NS * CHUNK)
RPT = N // NS
RCH = RPT // CHUNK
NB = 8

_F32 = jnp.float32


def _sc_scratch():
  s = [
      pltpu.VMEM((NCHT, CHUNK), jnp.int32),
      pltpu.VMEM((NCHT, CHUNK), jnp.int32),
      pltpu.VMEM_SHARED((N, H), _F32),
  ]
  s += [pltpu.VMEM((CHUNK, H), _F32)] * NB
  s += [pltpu.SemaphoreType.DMA] * NB
  s += [pltpu.SemaphoreType.DMA] * NB
  return s


def _zero_stripe(buf, accs, row0):
  zrow = jnp.zeros((H,), _F32)
  for i in range(CHUNK):
    buf[i] = zrow
  for acc in accs:
    for k in range(RCH):
      pltpu.sync_copy(buf, acc.at[pl.ds(row0 + k * CHUNK, CHUNK)])


def _scatter_loop(y_flat, sidx, didx, acc, bufs, gsem, ssem):
  for k in range(NB // 2):
    pltpu.async_copy(y_flat.at[sidx.at[k]], bufs[k], gsem[k])

  def step(i, _):
    for k in range(NB):
      j = NB * i + k
      pltpu.make_async_copy(y_flat.at[sidx.at[j]], bufs[k], gsem[k]).wait()
      pltpu.async_copy(bufs[k], acc.at[didx.at[j]], ssem[k], add=True)
      b2 = (k + NB // 2) % NB

      @pl.when(j + NB // 2 < NCHT)
      def _next():
        @pl.when(j >= NB // 2)
        def _drain():
          pltpu.make_async_copy(
              bufs[b2], acc.at[didx.at[j - NB // 2]], ssem[b2]).wait()

        pltpu.async_copy(
            y_flat.at[sidx.at[j + NB // 2]], bufs[b2], gsem[b2])

    return 0

  lax.fori_loop(0, NCHT // NB, step, 0)
  for k in range(NB):
    j = NCHT - NB + k
    pltpu.make_async_copy(bufs[k], acc.at[didx.at[j]], ssem[k]).wait()


_MESH = plsc.VectorSubcoreMesh(core_axis_name="c", subcore_axis_name="s")
_SC_PARAMS = pltpu.CompilerParams(use_tc_tiling_on_sc=False)


@functools.partial(
    pl.kernel, mesh=_MESH,
    out_type=[jax.ShapeDtypeStruct((NC, N, H), _F32),
              jax.ShapeDtypeStruct((NC, N, H), _F32)],
    scratch_types=_sc_scratch() +
    [pltpu.VMEM((CHUNK, H), _F32)] * NB +
    [pltpu.SemaphoreType.DMA] * NB + [
        pltpu.VMEM_SHARED((N, H), _F32),
    ],
    compiler_params=_SC_PARAMS)
def _sc1(y1, srcs, dsts, out_s, out_rcp, sidx, didx, acc, *rest):
  bufs = rest[:NB]
  gsem = rest[NB:2 * NB]
  ssem = rest[2 * NB:3 * NB]
  ones = rest[3 * NB:4 * NB]
  csem = rest[4 * NB:5 * NB]
  acc_c = rest[5 * NB]

  c = lax.axis_index("c")
  s = lax.axis_index("s")
  row0 = s * RPT

  _zero_stripe(bufs[0], [acc, acc_c], row0)
  orow = jnp.ones((H,), _F32)
  for b in range(NB):
    for i in range(CHUNK):
      ones[b][i] = orow
  plsc.subcore_barrier()

  ebase = s * NCHT
  pltpu.sync_copy(srcs.at[c, pl.ds(ebase, NCHT)], sidx)
  pltpu.sync_copy(dsts.at[c, pl.ds(ebase, NCHT)], didx)

  def cnt_step(i, _):
    for k in range(NB):
      j = NB * i + k

      @pl.when(j >= NB)
      def _drain():
        pltpu.make_async_copy(
            ones[k], acc_c.at[didx.at[j - NB]], csem[k]).wait()

      pltpu.async_copy(ones[k], acc_c.at[didx.at[j]], csem[k], add=True)
    return 0

  lax.fori_loop(0, NCHT // NB, cnt_step, 0)
  _scatter_loop(y1, sidx, didx, acc, bufs, gsem, ssem)
  for k in range(NB):
    pltpu.make_async_copy(
        ones[k], acc_c.at[didx.at[NCHT - NB + k]], csem[k]).wait()
  plsc.subcore_barrier()

  for k in range(RCH):
    r = row0 + k * CHUNK
    pltpu.sync_copy(acc.at[pl.ds(r, CHUNK)], bufs[0])
    pltpu.sync_copy(bufs[0], out_s.at[c, pl.ds(r, CHUNK)])
    pltpu.sync_copy(acc_c.at[pl.ds(r, CHUNK)], bufs[1])
    pltpu.sync_copy(bufs[1], out_rcp.at[c, pl.ds(r, CHUNK)])


@functools.partial(
    pl.kernel, mesh=_MESH,
    out_type=[jax.ShapeDtypeStruct((NC, N, H), _F32)],
    scratch_types=_sc_scratch() + [
        pltpu.VMEM((CHUNK, H), _F32),
        pltpu.VMEM((CHUNK, H), _F32),
        pltpu.VMEM((H,), _F32),
    ],
    compiler_params=_SC_PARAMS)
def _sc2(y2, srcs, dsts, rcp, r2, b2, out2, sidx, didx, acc,
         *rest):
  bufs = rest[:NB]
  gsem = rest[NB:2 * NB]
  ssem = rest[2 * NB:3 * NB]
  rbuf = rest[3 * NB]
  tbuf = rest[3 * NB + 1]
  bbuf = rest[3 * NB + 2]

  c = lax.axis_index("c")
  s = lax.axis_index("s")
  row0 = s * RPT

  _zero_stripe(bufs[0], [acc], row0)
  plsc.subcore_barrier()

  ebase = s * NCHT
  pltpu.sync_copy(srcs.at[c, pl.ds(ebase, NCHT)], sidx)
  pltpu.sync_copy(dsts.at[c, pl.ds(ebase, NCHT)], didx)
  pltpu.sync_copy(b2.at[c], bbuf)
  _scatter_loop(y2, sidx, didx, acc, bufs, gsem, ssem)
  plsc.subcore_barrier()

  bias = bbuf[...]
  for k in range(RCH):
    r = row0 + k * CHUNK
    pltpu.sync_copy(acc.at[pl.ds(r, CHUNK)], bufs[0])
    pltpu.sync_copy(rcp.at[c, pl.ds(r, CHUNK)], rbuf)
    pltpu.sync_copy(r2.at[c, pl.ds(r, CHUNK)], tbuf)

    def ep_row(i, _):
      bufs[0][i] = jnp.maximum(
          bufs[0][i] * rbuf[i] + bias + tbuf[i], 0.0)
      return 0

    lax.fori_loop(0, CHUNK, ep_row, 0)

    pltpu.sync_copy(bufs[0], out2.at[c, pl.ds(r, CHUNK)])


_R = 2000


def _tc1_body(xp, xu, wvl, wrl, wvr, wrr, y1, r1):
  p = xp[...]
  u = xu[...]
  y1[0] = jnp.dot(p, wvl[...], preferred_element_type=_F32)
  y1[1] = jnp.dot(u, wrl[...], preferred_element_type=_F32)
  r1[0] = jnp.dot(u, wvr[...], preferred_element_type=_F32)
  r1[1] = jnp.dot(p, wrr[...], preferred_element_type=_F32)


def _tc1(xp, xu, wvl, wrl, wvr, wrr):
  bx = pl.BlockSpec((_R, D), lambda i: (i, 0))
  bw = pl.BlockSpec((D, H), lambda i: (0, 0))
  bo = pl.BlockSpec((NC, _R, H), lambda i: (0, i, 0))
  return pl.pallas_call(
      _tc1_body,
      grid=(N // _R,),
      in_specs=[bx, bx, bw, bw, bw, bw],
      out_specs=[bo, bo],
      out_shape=[jax.ShapeDtypeStruct((NC, N, H), _F32)] * 2,
  )(xp, xu, wvl, wrl, wvr, wrr)


def _tc2_body(s1, cnt, r1, b1, wvl, wrl, wvr, wrr, y2, r2, rcp):
  sa = s1[...]
  ra = 1.0 / jnp.maximum(cnt[...], 1.0)
  oa = r1[...]
  ba = b1[...]
  u1 = jnp.maximum(sa[0] * ra[0] + ba[0] + oa[0], 0.0)
  p1 = jnp.maximum(sa[1] * ra[1] + ba[1] + oa[1], 0.0)
  y2[0] = jnp.dot(p1, wvl[...], preferred_element_type=_F32)
  y2[1] = jnp.dot(u1, wrl[...], preferred_element_type=_F32)
  r2[0] = jnp.dot(u1, wvr[...], preferred_element_type=_F32)
  r2[1] = jnp.dot(p1, wrr[...], preferred_element_type=_F32)
  rcp[...] = ra


def _tc2(s1, rcp, r1, b1, wvl, wrl, wvr, wrr):
  bp = pl.BlockSpec((NC, _R, H), lambda i: (0, i, 0))
  bb = pl.BlockSpec((NC, H), lambda i: (0, 0))
  bw = pl.BlockSpec((H, H), lambda i: (0, 0))
  return pl.pallas_call(
      _tc2_body,
      grid=(N // _R,),
      in_specs=[bp, bp, bp, bb, bw, bw, bw, bw],
      out_specs=[bp, bp, bp],
      out_shape=[jax.ShapeDtypeStruct((NC, N, H), _F32)] * 3,
  )(s1, rcp, r1, b1, wvl, wrl, wvr, wrr)


def kernel(x_Persona, x_Ubicacion, edge_index_visits, edge_index_rev,
           W1v_l, b1v, W1v_r, W1r_l, b1r, W1r_r,
           W2v_l, b2v, W2v_r, W2r_l, b2r, W2r_r):
  nch = E // CHUNK
  srcs = jnp.stack([edge_index_visits[0].reshape(nch, CHUNK),
                    edge_index_rev[0].reshape(nch, CHUNK) + N])
  dsts = jnp.stack([edge_index_visits[1].reshape(nch, CHUNK),
                    edge_index_rev[1].reshape(nch, CHUNK)])
  b1 = jnp.stack([b1v, b1r])
  b2 = jnp.stack([b2v, b2r])

  y1, r1 = _tc1(x_Persona, x_Ubicacion, W1v_l, W1r_l, W1v_r, W1r_r)
  s1, cnt = _sc1(y1.reshape(NC * N, H), srcs, dsts)
  y2, r2, rcp = _tc2(s1, cnt, r1, b1, W2v_l, W2r_l, W2v_r, W2r_r)
  out2 = _sc2(y2.reshape(NC * N, H), srcs, dsts, rcp, r2, b2)[0]
  return (out2[1], out2[0])
